# Initial kernel scaffold; baseline (speedup 1.0000x reference)
import jax
import jax.numpy as jnp
from jax import lax
from jax.experimental import pallas as pl
from jax.experimental.pallas import tpu as pltpu
from jax.experimental.pallas import tpu_sc as plsc

_BATCH = 16384
_FIELDS = 26
_DIM = 16
_NC = 2
_NS = 16
_NW = _NC * _NS
_ROWS_PER_W = _BATCH // _NW
_CHUNK = 64
_NCHUNK = _ROWS_PER_W // _CHUNK
_IPC = _CHUNK * _FIELDS
_GS = 128
_NG = _IPC // _GS


def _fm_body(idx_hbm, val_hbm, emb_hbm, w1_hbm, bias_hbm, out_hbm,
             idx_v, val_v, emb_v, w_v, out_v, bias_v, sem):
    wid = lax.axis_index("s") * _NC + lax.axis_index("c")
    pltpu.sync_copy(bias_hbm, bias_v)
    lane = lax.iota(jnp.int32, 16)

    @pl.loop(0, _NCHUNK)
    def _chunk(c):
        row0 = wid * _ROWS_PER_W + c * _CHUNK
        off = row0 * _FIELDS
        pltpu.sync_copy(idx_hbm.at[pl.ds(off, _IPC)], idx_v)
        pltpu.sync_copy(val_hbm.at[pl.ds(off, _IPC)], val_v)
        copies = []
        for g in range(_NG):
            isl = idx_v.at[pl.ds(g * _GS, _GS)]
            copies.append(pltpu.async_copy(
                emb_hbm.at[isl], emb_v.at[pl.ds(g * _GS, _GS)], sem))
            copies.append(pltpu.async_copy(
                w1_hbm.at[isl], w_v.at[pl.ds(g * _GS, _GS)], sem))
        for cp in copies:
            cp.wait()

        bias_vec = bias_v[...]
        for g in range(_CHUNK // 16):
            def _row(rr, carry, _g=g):
                acc_so, acc_fo = carry
                jbase = (_g * 16 + rr) * _FIELDS
                s = jnp.zeros((16,), jnp.float32)
                sq = jnp.zeros((16,), jnp.float32)
                fo = jnp.zeros((16,), jnp.float32)
                for f in range(_FIELDS):
                    jv = jnp.full((16,), jbase + f, jnp.int32)
                    xb = plsc.load_gather(val_v, [jv])
                    wb = plsc.load_gather(w_v, [jv])
                    row = plsc.load_gather(emb_v, [jv, lane])
                    ev = row * xb
                    s = s + ev
                    sq = sq + ev * ev
                    fo = fo + xb * wb
                red = jnp.sum(s * s - sq)
                m = lane == rr
                acc_so = jnp.where(m, red, acc_so)
                acc_fo = jnp.where(m, fo, acc_fo)
                return acc_so, acc_fo

            zero = jnp.zeros((16,), jnp.float32)
            acc_so, acc_fo = lax.fori_loop(0, 16, _row, (zero, zero))
            logit = bias_vec + acc_fo + 0.5 * acc_so
            out_v[pl.ds(g * 16, 16)] = 1.0 / (1.0 + jnp.exp(-logit))
        pltpu.sync_copy(out_v, out_hbm.at[pl.ds(row0, _CHUNK)])


@jax.jit
def _fm_sc(idx_flat, val_flat, emb_table, w1_flat, bias_vec):
    mesh = plsc.VectorSubcoreMesh(core_axis_name="c", subcore_axis_name="s")
    return pl.kernel(
        _fm_body,
        out_type=jax.ShapeDtypeStruct((_BATCH,), jnp.float32),
        mesh=mesh,
        scratch_types=[
            pltpu.VMEM((_IPC,), jnp.int32),
            pltpu.VMEM((_IPC,), jnp.float32),
            pltpu.VMEM((_IPC, _DIM), jnp.float32),
            pltpu.VMEM((_IPC,), jnp.float32),
            pltpu.VMEM((_CHUNK,), jnp.float32),
            pltpu.VMEM((16,), jnp.float32),
            pltpu.SemaphoreType.DMA,
        ],
    )(idx_flat, val_flat, emb_table, w1_flat, bias_vec)


def kernel(feat_index, feat_value, emb_table, w1, bias):
    idx_flat = feat_index.reshape(-1).astype(jnp.int32)
    val_flat = feat_value.reshape(-1)
    w1_flat = w1.reshape(-1)
    bias_vec = jnp.broadcast_to(jnp.asarray(bias, jnp.float32), (16,))
    return _fm_sc(idx_flat, val_flat, emb_table, w1_flat, bias_vec)


# speedup vs baseline: 1.3161x; 1.3161x over previous
import jax
import jax.numpy as jnp
from jax import lax
from jax.experimental import pallas as pl
from jax.experimental.pallas import tpu as pltpu
from jax.experimental.pallas import tpu_sc as plsc

_BATCH = 16384
_FIELDS = 26
_DIM = 16
_NC = 2
_NS = 16
_NW = _NC * _NS
_ROWS_PER_W = _BATCH // _NW
_CHUNK = 64
_NCHUNK = _ROWS_PER_W // _CHUNK
_IPC = _CHUNK * _FIELDS
_GS = 128
_NG = _IPC // _GS


def _fm_body(idx_hbm, val_hbm, emb_hbm, w1_hbm, bias_hbm, out_hbm,
             idx_v, val_v, emb_v, w_v, out_v, bias_v, sem):
    wid = lax.axis_index("s") * _NC + lax.axis_index("c")
    pltpu.sync_copy(bias_hbm, bias_v)
    lane = lax.iota(jnp.int32, 16)

    @pl.loop(0, _NCHUNK)
    def _chunk(c):
        row0 = wid * _ROWS_PER_W + c * _CHUNK
        off = row0 * _FIELDS
        pltpu.sync_copy(idx_hbm.at[pl.ds(off, _IPC)], idx_v)
        pltpu.sync_copy(val_hbm.at[pl.ds(off, _IPC)], val_v)
        copies = []
        for g in range(_NG):
            isl = idx_v.at[pl.ds(g * _GS, _GS)]
            copies.append(pltpu.async_copy(
                emb_hbm.at[isl], emb_v.at[pl.ds(g * _GS, _GS)], sem))
            copies.append(pltpu.async_copy(
                w1_hbm.at[isl], w_v.at[pl.ds(g * _GS, _GS)], sem))
        for cp in copies:
            cp.wait()

        bias_vec = bias_v[...]
        himask = lane >= (2 * 16 - _FIELDS)
        for g in range(_CHUNK // 16):
            def _row(rr, acc, _g=g):
                jbase = (_g * 16 + rr) * _FIELDS
                xv0 = val_v[pl.ds(jbase, 16)]
                xv1 = val_v[pl.ds(jbase + _FIELDS - 16, 16)]
                wv0 = w_v[pl.ds(jbase, 16)]
                wv1 = w_v[pl.ds(jbase + _FIELDS - 16, 16)]
                fo_vec = xv0 * wv0 + jnp.where(himask, xv1 * wv1, 0.0)
                s = jnp.zeros((16,), jnp.float32)
                sq = jnp.zeros((16,), jnp.float32)
                for f in range(_FIELDS):
                    x = xv0[f] if f < 16 else xv1[f - (_FIELDS - 16)]
                    xb = jnp.full((16,), x, jnp.float32)
                    row = emb_v[jbase + f, :]
                    ev = row * xb
                    s = s + ev
                    sq = sq + ev * ev
                red = jnp.sum(fo_vec + 0.5 * (s * s - sq))
                return jnp.where(lane == rr, red, acc)

            acc = lax.fori_loop(0, 16, _row, jnp.zeros((16,), jnp.float32))
            logit = bias_vec + acc
            out_v[pl.ds(g * 16, 16)] = 1.0 / (1.0 + jnp.exp(-logit))
        pltpu.sync_copy(out_v, out_hbm.at[pl.ds(row0, _CHUNK)])


@jax.jit
def _fm_sc(idx_flat, val_flat, emb_table, w1_flat, bias_vec):
    mesh = plsc.VectorSubcoreMesh(core_axis_name="c", subcore_axis_name="s")
    return pl.kernel(
        _fm_body,
        out_type=jax.ShapeDtypeStruct((_BATCH,), jnp.float32),
        mesh=mesh,
        compiler_params=pltpu.CompilerParams(
            needs_layout_passes=False, use_tc_tiling_on_sc=False),
        scratch_types=[
            pltpu.VMEM((_IPC,), jnp.int32),
            pltpu.VMEM((_IPC,), jnp.float32),
            pltpu.VMEM((_IPC, _DIM), jnp.float32),
            pltpu.VMEM((_IPC,), jnp.float32),
            pltpu.VMEM((_CHUNK,), jnp.float32),
            pltpu.VMEM((16,), jnp.float32),
            pltpu.SemaphoreType.DMA,
        ],
    )(idx_flat, val_flat, emb_table, w1_flat, bias_vec)


def kernel(feat_index, feat_value, emb_table, w1, bias):
    idx_flat = feat_index.reshape(-1).astype(jnp.int32)
    val_flat = feat_value.reshape(-1)
    w1_flat = w1.reshape(-1)
    bias_vec = jnp.broadcast_to(jnp.asarray(bias, jnp.float32), (16,))
    return _fm_sc(idx_flat, val_flat, emb_table, w1_flat, bias_vec)


# speedup vs baseline: 1.3453x; 1.0221x over previous
import jax
import jax.numpy as jnp
from jax import lax
from jax.experimental import pallas as pl
from jax.experimental.pallas import tpu as pltpu
from jax.experimental.pallas import tpu_sc as plsc

_BATCH = 16384
_FIELDS = 26
_DIM = 16
_NC = 2
_NS = 16
_NW = _NC * _NS
_ROWS_PER_W = _BATCH // _NW
_CHUNK = 64
_NCHUNK = _ROWS_PER_W // _CHUNK
_IPC = _CHUNK * _FIELDS
_GS = 128
_NG = _IPC // _GS


def _fm_body(idx_hbm, val_hbm, emb_hbm, w1_hbm, bias_hbm, out_hbm,
             idx_v, val_v, emb_v, w_v, out_v, bias_v, sem):
    wid = lax.axis_index("s") * _NC + lax.axis_index("c")
    pltpu.sync_copy(bias_hbm, bias_v)
    lane = lax.iota(jnp.int32, 16)
    bias_vec = bias_v[...]
    himask = lane >= (2 * 16 - _FIELDS)

    def _stage(c, p):
        off = (wid * _ROWS_PER_W + c * _CHUNK) * _FIELDS
        pltpu.sync_copy(idx_hbm.at[pl.ds(off, _IPC)], idx_v.at[p])
        pltpu.sync_copy(val_hbm.at[pl.ds(off, _IPC)], val_v.at[p])
        copies = []
        for g in range(_NG):
            isl = idx_v.at[p, pl.ds(g * _GS, _GS)]
            copies.append(pltpu.async_copy(
                emb_hbm.at[isl], emb_v.at[p, pl.ds(g * _GS, _GS)], sem))
            copies.append(pltpu.async_copy(
                w1_hbm.at[isl], w_v.at[p, pl.ds(g * _GS, _GS)], sem))
        return copies

    def _compute(c, p):
        row0 = wid * _ROWS_PER_W + c * _CHUNK

        def _group(g, carry):
            def _row(rr, acc):
                jbase = (g * 16 + rr) * _FIELDS
                xv0 = val_v[p, pl.ds(jbase, 16)]
                xv1 = val_v[p, pl.ds(jbase + _FIELDS - 16, 16)]
                wv0 = w_v[p, pl.ds(jbase, 16)]
                wv1 = w_v[p, pl.ds(jbase + _FIELDS - 16, 16)]
                fo_vec = xv0 * wv0 + jnp.where(himask, xv1 * wv1, 0.0)
                s = jnp.zeros((16,), jnp.float32)
                sq = jnp.zeros((16,), jnp.float32)
                for f in range(_FIELDS):
                    x = xv0[f] if f < 16 else xv1[f - (_FIELDS - 16)]
                    xb = jnp.full((16,), x, jnp.float32)
                    row = emb_v[p, jbase + f, :]
                    ev = row * xb
                    s = s + ev
                    sq = sq + ev * ev
                red = jnp.sum(fo_vec + 0.5 * (s * s - sq))
                return jnp.where(lane == rr, red, acc)

            acc = lax.fori_loop(0, 16, _row, jnp.zeros((16,), jnp.float32))
            logit = bias_vec + acc
            out_v[pl.ds(g * 16, 16)] = 1.0 / (1.0 + jnp.exp(-logit))
            return carry

        lax.fori_loop(0, _CHUNK // 16, _group, jnp.int32(0))
        pltpu.sync_copy(out_v, out_hbm.at[pl.ds(row0, _CHUNK)])

    inflight = _stage(0, 0)
    for c in range(_NCHUNK):
        for cp in inflight:
            cp.wait()
        if c + 1 < _NCHUNK:
            nxt = _stage(c + 1, (c + 1) % 2)
        else:
            nxt = []
        _compute(c, c % 2)
        inflight = nxt


@jax.jit
def _fm_sc(idx_flat, val_flat, emb_table, w1_flat, bias_vec):
    mesh = plsc.VectorSubcoreMesh(core_axis_name="c", subcore_axis_name="s")
    return pl.kernel(
        _fm_body,
        out_type=jax.ShapeDtypeStruct((_BATCH,), jnp.float32),
        mesh=mesh,
        compiler_params=pltpu.CompilerParams(
            needs_layout_passes=False, use_tc_tiling_on_sc=False),
        scratch_types=[
            pltpu.VMEM((2, _IPC), jnp.int32),
            pltpu.VMEM((2, _IPC), jnp.float32),
            pltpu.VMEM((2, _IPC, _DIM), jnp.float32),
            pltpu.VMEM((2, _IPC), jnp.float32),
            pltpu.VMEM((_CHUNK,), jnp.float32),
            pltpu.VMEM((16,), jnp.float32),
            pltpu.SemaphoreType.DMA,
        ],
    )(idx_flat, val_flat, emb_table, w1_flat, bias_vec)


def kernel(feat_index, feat_value, emb_table, w1, bias):
    idx_flat = feat_index.reshape(-1).astype(jnp.int32)
    val_flat = feat_value.reshape(-1)
    w1_flat = w1.reshape(-1)
    bias_vec = jnp.broadcast_to(jnp.asarray(bias, jnp.float32), (16,))
    return _fm_sc(idx_flat, val_flat, emb_table, w1_flat, bias_vec)
